# baseline (speedup 1.0000x reference)
import jax
import jax.numpy as jnp
from jax import lax
from jax.experimental import pallas as pl
from jax.experimental.pallas import tpu as pltpu
from jax.experimental.pallas import tpu_sc as plsc

N = 10000
NPAD = 10240
E = 160000
IN = 256
OUT = 256
EIN = 16

NC = 2
NS = 16
NW = NC * NS

BLKE = 1280
EC1 = 64 * BLKE
EC2 = E - EC1

BLK = 128
ZROWS = NPAD // NS

_INV_SQRT_BN = (1.0 + 1e-5) ** -0.5


def _leaky(x):
    return jnp.where(x > 0, x, 0.1 * x)


def _mesh():
    return plsc.VectorSubcoreMesh(core_axis_name="c", subcore_axis_name="s",
                                  num_cores=NC, num_subcores=NS)


def _node_embed_body(h_ref, w_ref, o_ref):
    o_ref[...] = jnp.dot(h_ref[...], w_ref[...],
                         preferred_element_type=jnp.float32)


def _node_embed(h_pad, W):
    blk = 512
    grid = NPAD // blk
    return pl.pallas_call(
        _node_embed_body,
        grid=(grid,),
        in_specs=[
            pl.BlockSpec((blk, IN), lambda i: (i, 0)),
            pl.BlockSpec((IN, OUT), lambda i: (0, 0)),
        ],
        out_specs=pl.BlockSpec((blk, OUT), lambda i: (i, 0)),
        out_shape=jax.ShapeDtypeStruct((NPAD, OUT), jnp.float32),
    )(h_pad, W)


def _make_gather_hn(ec, name):
    per_w = ec // NW
    assert per_w % 8 == 0
    nfull = per_w // BLK
    tail = per_w - nfull * BLK
    ng = nfull // 3
    leftover = nfull % 3

    def body(hn_hbm, src_hbm, dst_hbm, hs_hbm, hd_hbm, *scr):
        if tail:
            idx_full, r0, r1, r2, rt, sem = scr
        else:
            idx_full, r0, r1, r2, sem = scr
        wid = lax.axis_index("s") * NC + lax.axis_index("c")
        base = wid * per_w
        rows = (r0, r1, r2)

        def one_pass(which_idx, which_out):
            pltpu.sync_copy(which_idx.at[pl.ds(base, per_w)], idx_full)

            def do_set(goff, nk):
                hs = []
                for k in range(nk):
                    isl = idx_full.at[pl.ds(goff + k * BLK, BLK)]
                    cp = pltpu.make_async_copy(hn_hbm.at[isl], rows[k], sem)
                    cp.start()
                    hs.append(cp)
                for cp in hs:
                    cp.wait()
                ws = []
                for k in range(nk):
                    cp = pltpu.make_async_copy(
                        rows[k],
                        which_out.at[pl.ds(base + goff + k * BLK, BLK)], sem)
                    cp.start()
                    ws.append(cp)
                for cp in ws:
                    cp.wait()

            def group(g, carry):
                do_set(g * (3 * BLK), 3)
                return carry

            lax.fori_loop(0, ng, group, 0)
            if leftover:
                do_set(ng * 3 * BLK, leftover)
            if tail:
                toff = nfull * BLK
                isl = idx_full.at[pl.ds(toff, tail)]
                pltpu.sync_copy(hn_hbm.at[isl], rt)
                pltpu.sync_copy(rt, which_out.at[pl.ds(base + toff, tail)])

        one_pass(src_hbm, hs_hbm)
        one_pass(dst_hbm, hd_hbm)

    scratch = [pltpu.VMEM((per_w,), jnp.int32)]
    scratch += [pltpu.VMEM((BLK, OUT), jnp.float32)] * 3
    if tail:
        scratch += [pltpu.VMEM((tail, OUT), jnp.float32)]
    scratch += [pltpu.SemaphoreType.DMA]

    return pl.kernel(
        body,
        name=name,
        out_type=[jax.ShapeDtypeStruct((ec, OUT), jnp.float32),
                  jax.ShapeDtypeStruct((ec, OUT), jnp.float32)],
        mesh=_mesh(),
        scratch_types=scratch,
    )


def _gather_hn_1(hn, src, dst):
    return _make_gather_hn(EC1, "sc_g1a")(hn, src, dst)


def _gather_hn_2(hn, src, dst):
    return _make_gather_hn(EC2, "sc_g1b")(hn, src, dst)


def _make_edge_body(aliased):
    def body(*refs):
        if aliased:
            (hs_ref, hd_ref, ew_ref, wf_ref, b_ref, wee_ref, wsf_ref,
             wfc_ref, waw_ref, wao_ref, _sat_prev, _wp_prev,
             sat_ref, p01_ref, p2_ref, wpart_ref) = refs
        else:
            (hs_ref, hd_ref, ew_ref, wf_ref, b_ref, wee_ref, wsf_ref,
             wfc_ref, waw_ref, wao_ref,
             sat_ref, p01_ref, p2_ref, wpart_ref) = refs
        hs = hs_ref[...]
        hd = hd_ref[...]
        ew = ew_ref[...]
        ee = jnp.dot(ew, wee_ref[...], preferred_element_type=jnp.float32)
        esa = jnp.dot(_leaky(ee), wsf_ref[...],
                      preferred_element_type=jnp.float32)
        w1 = esa * ee

        sat_ref[:, 0:OUT] = hs
        sat_ref[:, OUT:2 * OUT] = hd
        sat_ref[:, 2 * OUT:2 * OUT + EIN] = w1

        wf = wf_ref[...].astype(jnp.bfloat16)
        z2 = (jnp.dot(hs.astype(jnp.bfloat16), wf[0:OUT],
                      preferred_element_type=jnp.float32)
              + jnp.dot(hd.astype(jnp.bfloat16), wf[OUT:2 * OUT],
                        preferred_element_type=jnp.float32)
              + jnp.dot(w1.astype(jnp.bfloat16), wf[2 * OUT:2 * OUT + EIN],
                        preferred_element_type=jnp.float32)
              + b_ref[...])
        inw = _leaky(z2)
        att = jnp.dot(inw, wfc_ref[...], preferred_element_type=jnp.float32)
        ex = jnp.exp(att)
        w2 = att * w1

        exhs = ex * hs
        p01_ref[0, :, :] = exhs[:, 0:128]
        p01_ref[1, :, :] = exhs[:, 128:256]
        p2_ref[:, 0:EIN] = ex * w2
        p2_ref[:, EIN:EIN + 1] = ex
        p2_ref[:, EIN + 1:128] = jnp.zeros_like(ex) * jnp.zeros(
            (1, 127 - EIN), jnp.float32)

        wpart_ref[...] = (
            jnp.dot(w2 * _INV_SQRT_BN, waw_ref[...],
                    preferred_element_type=jnp.float32)
            + jnp.dot(ew, wao_ref[...], preferred_element_type=jnp.float32))

    return body


def _make_edge_chain(ec, blk_off, aliased):
    grid = ec // BLKE
    full = lambda r, c: pl.BlockSpec((r, c), lambda i: (0, 0))
    in_specs = [
        pl.BlockSpec((BLKE, OUT), lambda i: (i, 0)),
        pl.BlockSpec((BLKE, OUT), lambda i: (i, 0)),
        pl.BlockSpec((BLKE, EIN), lambda i: (i, 0)),
        full(2 * OUT + EIN, OUT),
        full(1, OUT),
        full(EIN, EIN),
        full(EIN, 1),
        full(OUT, 1),
        full(EIN, EIN),
        full(EIN, EIN),
    ]
    kwargs = {}
    if aliased:
        in_specs += [pl.BlockSpec(memory_space=pl.ANY),
                     pl.BlockSpec(memory_space=pl.ANY)]
        kwargs["input_output_aliases"] = {10: 0, 11: 3}
    return pl.pallas_call(
        _make_edge_body(aliased),
        grid=(grid,),
        in_specs=in_specs,
        out_specs=[
            pl.BlockSpec((BLKE, 2 * OUT + EIN), lambda i: (i + blk_off, 0)),
            pl.BlockSpec((2, BLKE, 128), lambda i: (0, i, 0)),
            pl.BlockSpec((BLKE, 128), lambda i: (i, 0)),
            pl.BlockSpec((BLKE, EIN), lambda i: (i + blk_off, 0)),
        ],
        out_shape=[
            jax.ShapeDtypeStruct((E, 2 * OUT + EIN), jnp.float32),
            jax.ShapeDtypeStruct((2, ec, 128), jnp.float32),
            jax.ShapeDtypeStruct((ec, 128), jnp.float32),
            jax.ShapeDtypeStruct((E, EIN), jnp.float32),
        ],
        **kwargs,
    )


def _make_segment_scatter(ec, name):
    per_t = ec // NS
    assert per_t % 8 == 0
    nfull_a = per_t // BLK
    tail_a = per_t - nfull_a * BLK
    ng_a = nfull_a // 2
    left_a = nfull_a % 2
    half = ec // NC
    per_tb = half // NS
    assert per_tb % 8 == 0
    nfull_b = per_tb // BLK
    tail_b = per_tb - nfull_b * BLK
    ng_b = nfull_b // 2
    left_b = nfull_b % 2

    def body(p01_hbm, p2_hbm, dst_hbm, zero_hbm, u01_hbm, u2_hbm, *scr):
        scr = list(scr)
        i0, i1, r0, r1 = scr[:4]
        scr = scr[4:]
        if tail_a:
            idx_ta, rows_ta = scr[:2]
            scr = scr[2:]
        if tail_b:
            idx_tb, rows_tb = scr[:2]
            scr = scr[2:]
        sem, shared = scr
        c = lax.axis_index("c")
        s = lax.axis_index("s")
        idxs = (i0, i1)
        rows = (r0, r1)

        def zero_my_slice():
            pltpu.sync_copy(zero_hbm, shared.at[pl.ds(s * ZROWS, ZROWS)])

        def do_group(off, nk, read_rows):
            hs = []
            for k in range(nk):
                cp = pltpu.make_async_copy(
                    dst_hbm.at[pl.ds(off + k * BLK, BLK)], idxs[k], sem)
                cp.start()
                hs.append(cp)
                hs.append(read_rows(off + k * BLK, rows[k]))
            for cp in hs:
                cp.wait()
            sc = []
            for k in range(nk):
                cp = pltpu.async_copy(rows[k], shared.at[idxs[k]], sem,
                                      add=True)
                sc.append(cp)
            for cp in sc:
                cp.wait()

        def do_tail(off, n, idx_ref, rows_ref, p_at):
            pltpu.sync_copy(dst_hbm.at[pl.ds(off, n)], idx_ref)
            pltpu.sync_copy(p_at(off, n), rows_ref)
            pltpu.sync_copy(rows_ref, shared.at[idx_ref], add=True)

        def read_a(off, rref):
            cp = pltpu.make_async_copy(p01_hbm.at[c, pl.ds(off, BLK)], rref,
                                       sem)
            cp.start()
            return cp

        def read_b(off, rref):
            cp = pltpu.make_async_copy(p2_hbm.at[pl.ds(off, BLK)], rref, sem)
            cp.start()
            return cp

        def writeout(u_hbm):
            pltpu.sync_copy(shared.at[pl.ds(s * ZROWS, ZROWS)],
                            u_hbm.at[c, pl.ds(s * ZROWS, ZROWS)])

        zero_my_slice()
        plsc.subcore_barrier()
        base = s * per_t

        def body_a(g, carry):
            do_group(base + g * (2 * BLK), 2, read_a)
            return carry

        lax.fori_loop(0, ng_a, body_a, 0)
        if left_a:
            do_group(base + ng_a * 2 * BLK, 1, read_a)
        if tail_a:
            do_tail(base + nfull_a * BLK, tail_a, idx_ta, rows_ta,
                    lambda off, n: p01_hbm.at[c, pl.ds(off, n)])
        plsc.subcore_barrier()
        writeout(u01_hbm)
        zero_my_slice()
        plsc.subcore_barrier()

        base_b = c * half + s * per_tb

        def body_b(g, carry):
            do_group(base_b + g * (2 * BLK), 2, read_b)
            return carry

        lax.fori_loop(0, ng_b, body_b, 0)
        if left_b:
            do_group(base_b + ng_b * 2 * BLK, 1, read_b)
        if tail_b:
            do_tail(base_b + nfull_b * BLK, tail_b, idx_tb, rows_tb,
                    lambda off, n: p2_hbm.at[pl.ds(off, n)])
        plsc.subcore_barrier()
        writeout(u2_hbm)

    scratch = [pltpu.VMEM((BLK,), jnp.int32)] * 2
    scratch += [pltpu.VMEM((BLK, 128), jnp.float32)] * 2
    if tail_a:
        scratch += [pltpu.VMEM((tail_a,), jnp.int32),
                    pltpu.VMEM((tail_a, 128), jnp.float32)]
    if tail_b:
        scratch += [pltpu.VMEM((tail_b,), jnp.int32),
                    pltpu.VMEM((tail_b, 128), jnp.float32)]
    scratch += [pltpu.SemaphoreType.DMA,
                pltpu.VMEM_SHARED((NPAD, 128), jnp.float32)]

    return pl.kernel(
        body,
        name=name,
        out_type=[jax.ShapeDtypeStruct((2, NPAD, 128), jnp.float32),
                  jax.ShapeDtypeStruct((2, NPAD, 128), jnp.float32)],
        mesh=_mesh(),
        scratch_types=scratch,
    )


def _segment_scatter_1(p01, p2, dst, zero_blk):
    return _make_segment_scatter(EC1, "sc_scat_a")(p01, p2, dst, zero_blk)


def _segment_scatter_2(p01, p2, dst, zero_blk):
    return _make_segment_scatter(EC2, "sc_scat_b")(p01, p2, dst, zero_blk)


def _node_update_body(ua_ref, ub_ref, va_ref, vb_ref, hn_ref, wc_ref,
                      bc_ref, was_ref, wad_ref, ho_ref, t_ref):
    u0 = ua_ref[0, :, :] + ub_ref[0, :, :]
    u1 = ua_ref[1, :, :] + ub_ref[1, :, :]
    u2 = va_ref[0, :, :] + va_ref[1, :, :] + vb_ref[0, :, :] + vb_ref[1, :, :]
    hn = hn_ref[...]
    denom = u2[:, EIN:EIN + 1]
    rcp = jnp.where(denom > 0, 1.0 / jnp.maximum(denom, 1e-12), 0.0)
    wc = wc_ref[...]
    h_new = (jnp.dot(u0 * rcp, wc[0:128], preferred_element_type=jnp.float32)
             + jnp.dot(u1 * rcp, wc[128:256],
                       preferred_element_type=jnp.float32)
             + jnp.dot(u2[:, 0:EIN] * rcp, wc[256:272],
                       preferred_element_type=jnp.float32)
             + jnp.dot(hn, wc[272:272 + OUT],
                       preferred_element_type=jnp.float32)
             + bc_ref[...])
    h_out = jnp.where(denom > 0, h_new, hn)
    ho_ref[...] = h_out
    t_ref[:, 0:EIN] = jnp.dot(h_out, was_ref[...],
                              preferred_element_type=jnp.float32)
    t_ref[:, EIN:2 * EIN] = jnp.dot(h_out, wad_ref[...],
                                    preferred_element_type=jnp.float32)
    t_ref[:, 2 * EIN:128] = jnp.zeros_like(t_ref[:, 2 * EIN:128])


def _node_update(u01a, u01b, u2a, u2b, hn, Wc, b_c, Was, Wad):
    blk = 512
    grid = NPAD // blk
    full = lambda r, c: pl.BlockSpec((r, c), lambda i: (0, 0))
    stk = pl.BlockSpec((2, blk, 128), lambda i: (0, i, 0))
    return pl.pallas_call(
        _node_update_body,
        grid=(grid,),
        in_specs=[
            stk, stk, stk, stk,
            pl.BlockSpec((blk, OUT), lambda i: (i, 0)),
            full(2 * OUT + EIN, OUT),
            full(1, OUT),
            full(OUT, EIN),
            full(OUT, EIN),
        ],
        out_specs=[
            pl.BlockSpec((blk, OUT), lambda i: (i, 0)),
            pl.BlockSpec((blk, 128), lambda i: (i, 0)),
        ],
        out_shape=[
            jax.ShapeDtypeStruct((NPAD, OUT), jnp.float32),
            jax.ShapeDtypeStruct((NPAD, 128), jnp.float32),
        ],
    )(u01a, u01b, u2a, u2b, hn, Wc, b_c, Was, Wad)


_G2_PER_W = E // NW
_G2_NFULL = _G2_PER_W // BLK
_G2_TAIL = _G2_PER_W - _G2_NFULL * BLK
_G2_NG = 19


def _gather_pq_body(t_hbm, src_hbm, dst_hbm, wp_hbm, wout_hbm,
                    si_full, di_full,
                    rs0, rs1, rd0, rd1,
                    wp0, wp1,
                    rs_t, rd_t, wp_t, sem):
    wid = lax.axis_index("s") * NC + lax.axis_index("c")
    base = wid * _G2_PER_W
    rss = (rs0, rs1)
    rds = (rd0, rd1)
    wps = (wp0, wp1)

    pltpu.sync_copy(src_hbm.at[pl.ds(base, _G2_PER_W)], si_full)
    pltpu.sync_copy(dst_hbm.at[pl.ds(base, _G2_PER_W)], di_full)

    def compute_rows(n, rs_ref, rd_ref, wp_ref):
        def row(j, carry):
            wp_ref[j, :] = (rs_ref[j, pl.ds(0, EIN)]
                            + rd_ref[j, pl.ds(EIN, EIN)]
                            + wp_ref[j, :])
            return carry

        lax.fori_loop(0, n, row, 0)

    def do_set(goff, nk):
        hs = []
        for k in range(nk):
            loc = goff + k * BLK
            for cp in (
                pltpu.make_async_copy(
                    t_hbm.at[si_full.at[pl.ds(loc, BLK)]], rss[k], sem),
                pltpu.make_async_copy(
                    t_hbm.at[di_full.at[pl.ds(loc, BLK)]], rds[k], sem),
                pltpu.make_async_copy(
                    wp_hbm.at[pl.ds(base + loc, BLK)], wps[k], sem),
            ):
                cp.start()
                hs.append(cp)
        for cp in hs:
            cp.wait()
        for k in range(nk):
            compute_rows(BLK, rss[k], rds[k], wps[k])
        ws = []
        for k in range(nk):
            cp = pltpu.make_async_copy(
                wps[k], wout_hbm.at[pl.ds(base + goff + k * BLK, BLK)], sem)
            cp.start()
            ws.append(cp)
        for cp in ws:
            cp.wait()

    def group(g, carry):
        do_set(g * (2 * BLK), 2)
        return carry

    lax.fori_loop(0, _G2_NG, group, 0)
    do_set(_G2_NG * 2 * BLK, 1)
    toff = _G2_NFULL * BLK
    pltpu.sync_copy(t_hbm.at[si_full.at[pl.ds(toff, _G2_TAIL)]], rs_t)
    pltpu.sync_copy(t_hbm.at[di_full.at[pl.ds(toff, _G2_TAIL)]], rd_t)
    pltpu.sync_copy(wp_hbm.at[pl.ds(base + toff, _G2_TAIL)], wp_t)
    compute_rows(_G2_TAIL, rs_t, rd_t, wp_t)
    pltpu.sync_copy(wp_t, wout_hbm.at[pl.ds(base + toff, _G2_TAIL)])


def _gather_pq(T, src, dst, wpart):
    f = pl.kernel(
        _gather_pq_body,
        name="sc_g2",
        out_type=[jax.ShapeDtypeStruct((E, EIN), jnp.float32)],
        mesh=_mesh(),
        scratch_types=(
            [pltpu.VMEM((_G2_PER_W,), jnp.int32)] * 2
            + [pltpu.VMEM((BLK, 128), jnp.float32)] * 4
            + [pltpu.VMEM((BLK, EIN), jnp.float32)] * 2
            + [pltpu.VMEM((_G2_TAIL, 128), jnp.float32)] * 2
            + [pltpu.VMEM((_G2_TAIL, EIN), jnp.float32)] * 1
            + [pltpu.SemaphoreType.DMA]
        ),
    )
    (w_out,) = f(T, src, dst, wpart)
    return w_out


def kernel(h, edge_w, W_embed_node, W_attn_fc, W_inter_fuse, b_inter_fuse,
           W_embed_edge, W_edge_sf_atten, W_concentrate_h, b_concentrate_h,
           W_aggre, edge_index):
    src = edge_index[0].astype(jnp.int32)
    dst = edge_index[1].astype(jnp.int32)

    h_pad = jnp.pad(h, ((0, NPAD - N), (0, 0)))
    hn = _node_embed(h_pad, W_embed_node)

    src1, src2 = src[:EC1], src[EC1:]
    dst1, dst2 = dst[:EC1], dst[EC1:]
    hs1, hd1 = _gather_hn_1(hn, src1, dst1)
    hs2, hd2 = _gather_hn_2(hn, src2, dst2)

    b_if = b_inter_fuse.reshape(1, OUT)
    waw = W_aggre[2 * OUT:2 * OUT + EIN]
    wao = W_aggre[2 * OUT + EIN:]
    chain1 = _make_edge_chain(EC1, 0, True)
    chain2 = _make_edge_chain(EC2, EC1 // BLKE, True)
    sat0 = jnp.zeros((E, 2 * OUT + EIN), jnp.float32)
    wp0 = jnp.zeros((E, EIN), jnp.float32)
    sat1, p01_1, p2_1, wp1 = chain1(
        hs1, hd1, edge_w[:EC1], W_inter_fuse, b_if, W_embed_edge,
        W_edge_sf_atten, W_attn_fc, waw, wao, sat0, wp0)
    sat, p01_2, p2_2, wpart = chain2(
        hs2, hd2, edge_w[EC1:], W_inter_fuse, b_if, W_embed_edge,
        W_edge_sf_atten, W_attn_fc, waw, wao, sat1, wp1)

    zero_blk = jnp.zeros((ZROWS, 128), jnp.float32)
    u01a, u2a = _segment_scatter_1(p01_1, p2_1, dst1, zero_blk)
    u01b, u2b = _segment_scatter_2(p01_2, p2_2, dst2, zero_blk)

    h_out_pad, T = _node_update(
        u01a, u01b, u2a, u2b, hn, W_concentrate_h,
        b_concentrate_h.reshape(1, OUT), W_aggre[0:OUT], W_aggre[OUT:2 * OUT])

    w_out = _gather_pq(T, src, dst, wpart)

    return h_out_pad[:N], w_out, sat


# speedup vs baseline: 1.3940x; 1.3940x over previous
import jax
import jax.numpy as jnp
from jax import lax
from jax.experimental import pallas as pl
from jax.experimental.pallas import tpu as pltpu
from jax.experimental.pallas import tpu_sc as plsc

N = 10000
NPAD = 10240
E = 160000
IN = 256
OUT = 256
EIN = 16

NC = 2
NS = 16
NW = NC * NS

BLKE = 1280
EC1 = 64 * BLKE
EC2 = E - EC1

BLK = 128
ZROWS = NPAD // NS

_INV_SQRT_BN = (1.0 + 1e-5) ** -0.5


def _leaky(x):
    return jnp.where(x > 0, x, 0.1 * x)


def _mesh():
    return plsc.VectorSubcoreMesh(core_axis_name="c", subcore_axis_name="s",
                                  num_cores=NC, num_subcores=NS)


def _node_embed_body(h_ref, w_ref, o_ref):
    o_ref[...] = jnp.dot(h_ref[...], w_ref[...],
                         preferred_element_type=jnp.float32)


def _node_embed(h_pad, W):
    blk = 512
    grid = NPAD // blk
    return pl.pallas_call(
        _node_embed_body,
        grid=(grid,),
        in_specs=[
            pl.BlockSpec((blk, IN), lambda i: (i, 0)),
            pl.BlockSpec((IN, OUT), lambda i: (0, 0)),
        ],
        out_specs=pl.BlockSpec((blk, OUT), lambda i: (i, 0)),
        out_shape=jax.ShapeDtypeStruct((NPAD, OUT), jnp.float32),
    )(h_pad, W)


def _make_gather_hn(ec, name):
    per_w = ec // NW
    assert per_w % 8 == 0
    nfull = per_w // BLK
    tail = per_w - nfull * BLK
    ng = nfull // 3
    leftover = nfull % 3

    def body(hn_hbm, src_hbm, dst_hbm, hs_hbm, hd_hbm, *scr):
        if tail:
            idx_full, r0, r1, r2, rt, sem = scr
        else:
            idx_full, r0, r1, r2, sem = scr
        wid = lax.axis_index("s") * NC + lax.axis_index("c")
        base = wid * per_w
        rows = (r0, r1, r2)

        def one_pass(which_idx, which_out):
            pltpu.sync_copy(which_idx.at[pl.ds(base, per_w)], idx_full)

            def do_set(goff, nk):
                hs = []
                for k in range(nk):
                    isl = idx_full.at[pl.ds(goff + k * BLK, BLK)]
                    cp = pltpu.make_async_copy(hn_hbm.at[isl], rows[k], sem)
                    cp.start()
                    hs.append(cp)
                for cp in hs:
                    cp.wait()
                ws = []
                for k in range(nk):
                    cp = pltpu.make_async_copy(
                        rows[k],
                        which_out.at[pl.ds(base + goff + k * BLK, BLK)], sem)
                    cp.start()
                    ws.append(cp)
                for cp in ws:
                    cp.wait()

            def group(g, carry):
                do_set(g * (3 * BLK), 3)
                return carry

            lax.fori_loop(0, ng, group, 0)
            if leftover:
                do_set(ng * 3 * BLK, leftover)
            if tail:
                toff = nfull * BLK
                isl = idx_full.at[pl.ds(toff, tail)]
                pltpu.sync_copy(hn_hbm.at[isl], rt)
                pltpu.sync_copy(rt, which_out.at[pl.ds(base + toff, tail)])

        one_pass(src_hbm, hs_hbm)
        one_pass(dst_hbm, hd_hbm)

    scratch = [pltpu.VMEM((per_w,), jnp.int32)]
    scratch += [pltpu.VMEM((BLK, OUT), jnp.float32)] * 3
    if tail:
        scratch += [pltpu.VMEM((tail, OUT), jnp.float32)]
    scratch += [pltpu.SemaphoreType.DMA]

    return pl.kernel(
        body,
        name=name,
        out_type=[jax.ShapeDtypeStruct((ec, OUT), jnp.float32),
                  jax.ShapeDtypeStruct((ec, OUT), jnp.float32)],
        mesh=_mesh(),
        scratch_types=scratch,
    )


def _gather_hn_1(hn, src, dst):
    return _make_gather_hn(EC1, "sc_g1a")(hn, src, dst)


def _gather_hn_2(hn, src, dst):
    return _make_gather_hn(EC2, "sc_g1b")(hn, src, dst)


def _make_edge_body(aliased):
    def body(*refs):
        if aliased:
            (hs_ref, hd_ref, ew_ref, wf_ref, b_ref, wee_ref, wsf_ref,
             wfc_ref, waw_ref, wao_ref, _sat_prev, _wp_prev,
             sat_ref, p01_ref, p2_ref, wpart_ref) = refs
        else:
            (hs_ref, hd_ref, ew_ref, wf_ref, b_ref, wee_ref, wsf_ref,
             wfc_ref, waw_ref, wao_ref,
             sat_ref, p01_ref, p2_ref, wpart_ref) = refs
        hs = hs_ref[...]
        hd = hd_ref[...]
        ew = ew_ref[...]
        ee = jnp.dot(ew, wee_ref[...], preferred_element_type=jnp.float32)
        esa = jnp.dot(_leaky(ee), wsf_ref[...],
                      preferred_element_type=jnp.float32)
        w1 = esa * ee

        sat_ref[0:OUT, :] = hs.T
        sat_ref[OUT:2 * OUT, :] = hd.T
        sat_ref[2 * OUT:2 * OUT + EIN, :] = w1.T

        wf = wf_ref[...].astype(jnp.bfloat16)
        z2 = (jnp.dot(hs.astype(jnp.bfloat16), wf[0:OUT],
                      preferred_element_type=jnp.float32)
              + jnp.dot(hd.astype(jnp.bfloat16), wf[OUT:2 * OUT],
                        preferred_element_type=jnp.float32)
              + jnp.dot(w1.astype(jnp.bfloat16), wf[2 * OUT:2 * OUT + EIN],
                        preferred_element_type=jnp.float32)
              + b_ref[...])
        inw = _leaky(z2)
        att = jnp.dot(inw, wfc_ref[...], preferred_element_type=jnp.float32)
        ex = jnp.exp(att)
        w2 = att * w1

        exhs = ex * hs
        p01_ref[0, :, :] = exhs[:, 0:128]
        p01_ref[1, :, :] = exhs[:, 128:256]
        p2_ref[:, 0:EIN] = ex * w2
        p2_ref[:, EIN:EIN + 1] = ex
        p2_ref[:, EIN + 1:128] = jnp.zeros_like(ex) * jnp.zeros(
            (1, 127 - EIN), jnp.float32)

        wpart_ref[...] = (
            jnp.dot(w2 * _INV_SQRT_BN, waw_ref[...],
                    preferred_element_type=jnp.float32)
            + jnp.dot(ew, wao_ref[...], preferred_element_type=jnp.float32))

    return body


def _make_edge_chain(ec, blk_off, aliased):
    grid = ec // BLKE
    full = lambda r, c: pl.BlockSpec((r, c), lambda i: (0, 0))
    in_specs = [
        pl.BlockSpec((BLKE, OUT), lambda i: (i, 0)),
        pl.BlockSpec((BLKE, OUT), lambda i: (i, 0)),
        pl.BlockSpec((BLKE, EIN), lambda i: (i, 0)),
        full(2 * OUT + EIN, OUT),
        full(1, OUT),
        full(EIN, EIN),
        full(EIN, 1),
        full(OUT, 1),
        full(EIN, EIN),
        full(EIN, EIN),
    ]
    kwargs = {}
    if aliased:
        in_specs += [pl.BlockSpec(memory_space=pl.ANY),
                     pl.BlockSpec(memory_space=pl.ANY)]
        kwargs["input_output_aliases"] = {10: 0, 11: 3}
    return pl.pallas_call(
        _make_edge_body(aliased),
        grid=(grid,),
        in_specs=in_specs,
        out_specs=[
            pl.BlockSpec((2 * OUT + EIN, BLKE), lambda i: (0, i + blk_off)),
            pl.BlockSpec((2, BLKE, 128), lambda i: (0, i, 0)),
            pl.BlockSpec((BLKE, 128), lambda i: (i, 0)),
            pl.BlockSpec((BLKE, EIN), lambda i: (i + blk_off, 0)),
        ],
        out_shape=[
            jax.ShapeDtypeStruct((2 * OUT + EIN, E), jnp.float32),
            jax.ShapeDtypeStruct((2, ec, 128), jnp.float32),
            jax.ShapeDtypeStruct((ec, 128), jnp.float32),
            jax.ShapeDtypeStruct((E, EIN), jnp.float32),
        ],
        **kwargs,
    )


def _make_segment_scatter(ec, name):
    per_t = ec // NS
    assert per_t % 8 == 0
    nfull_a = per_t // BLK
    tail_a = per_t - nfull_a * BLK
    ng_a = nfull_a // 2
    left_a = nfull_a % 2
    half = ec // NC
    per_tb = half // NS
    assert per_tb % 8 == 0
    nfull_b = per_tb // BLK
    tail_b = per_tb - nfull_b * BLK
    ng_b = nfull_b // 2
    left_b = nfull_b % 2

    def body(p01_hbm, p2_hbm, dst_hbm, zero_hbm, u01_hbm, u2_hbm, *scr):
        scr = list(scr)
        i0, i1, r0, r1 = scr[:4]
        scr = scr[4:]
        if tail_a:
            idx_ta, rows_ta = scr[:2]
            scr = scr[2:]
        if tail_b:
            idx_tb, rows_tb = scr[:2]
            scr = scr[2:]
        sem, shared = scr
        c = lax.axis_index("c")
        s = lax.axis_index("s")
        idxs = (i0, i1)
        rows = (r0, r1)

        def zero_my_slice():
            pltpu.sync_copy(zero_hbm, shared.at[pl.ds(s * ZROWS, ZROWS)])

        def do_group(off, nk, read_rows):
            hs = []
            for k in range(nk):
                cp = pltpu.make_async_copy(
                    dst_hbm.at[pl.ds(off + k * BLK, BLK)], idxs[k], sem)
                cp.start()
                hs.append(cp)
                hs.append(read_rows(off + k * BLK, rows[k]))
            for cp in hs:
                cp.wait()
            sc = []
            for k in range(nk):
                cp = pltpu.async_copy(rows[k], shared.at[idxs[k]], sem,
                                      add=True)
                sc.append(cp)
            for cp in sc:
                cp.wait()

        def do_tail(off, n, idx_ref, rows_ref, p_at):
            pltpu.sync_copy(dst_hbm.at[pl.ds(off, n)], idx_ref)
            pltpu.sync_copy(p_at(off, n), rows_ref)
            pltpu.sync_copy(rows_ref, shared.at[idx_ref], add=True)

        def read_a(off, rref):
            cp = pltpu.make_async_copy(p01_hbm.at[c, pl.ds(off, BLK)], rref,
                                       sem)
            cp.start()
            return cp

        def read_b(off, rref):
            cp = pltpu.make_async_copy(p2_hbm.at[pl.ds(off, BLK)], rref, sem)
            cp.start()
            return cp

        def writeout(u_hbm):
            pltpu.sync_copy(shared.at[pl.ds(s * ZROWS, ZROWS)],
                            u_hbm.at[c, pl.ds(s * ZROWS, ZROWS)])

        zero_my_slice()
        plsc.subcore_barrier()
        base = s * per_t

        def body_a(g, carry):
            do_group(base + g * (2 * BLK), 2, read_a)
            return carry

        lax.fori_loop(0, ng_a, body_a, 0)
        if left_a:
            do_group(base + ng_a * 2 * BLK, 1, read_a)
        if tail_a:
            do_tail(base + nfull_a * BLK, tail_a, idx_ta, rows_ta,
                    lambda off, n: p01_hbm.at[c, pl.ds(off, n)])
        plsc.subcore_barrier()
        writeout(u01_hbm)
        zero_my_slice()
        plsc.subcore_barrier()

        base_b = c * half + s * per_tb

        def body_b(g, carry):
            do_group(base_b + g * (2 * BLK), 2, read_b)
            return carry

        lax.fori_loop(0, ng_b, body_b, 0)
        if left_b:
            do_group(base_b + ng_b * 2 * BLK, 1, read_b)
        if tail_b:
            do_tail(base_b + nfull_b * BLK, tail_b, idx_tb, rows_tb,
                    lambda off, n: p2_hbm.at[pl.ds(off, n)])
        plsc.subcore_barrier()
        writeout(u2_hbm)

    scratch = [pltpu.VMEM((BLK,), jnp.int32)] * 2
    scratch += [pltpu.VMEM((BLK, 128), jnp.float32)] * 2
    if tail_a:
        scratch += [pltpu.VMEM((tail_a,), jnp.int32),
                    pltpu.VMEM((tail_a, 128), jnp.float32)]
    if tail_b:
        scratch += [pltpu.VMEM((tail_b,), jnp.int32),
                    pltpu.VMEM((tail_b, 128), jnp.float32)]
    scratch += [pltpu.SemaphoreType.DMA,
                pltpu.VMEM_SHARED((NPAD, 128), jnp.float32)]

    return pl.kernel(
        body,
        name=name,
        out_type=[jax.ShapeDtypeStruct((2, NPAD, 128), jnp.float32),
                  jax.ShapeDtypeStruct((2, NPAD, 128), jnp.float32)],
        mesh=_mesh(),
        scratch_types=scratch,
    )


def _segment_scatter_1(p01, p2, dst, zero_blk):
    return _make_segment_scatter(EC1, "sc_scat_a")(p01, p2, dst, zero_blk)


def _segment_scatter_2(p01, p2, dst, zero_blk):
    return _make_segment_scatter(EC2, "sc_scat_b")(p01, p2, dst, zero_blk)


def _node_update_body(ua_ref, ub_ref, va_ref, vb_ref, hn_ref, wc_ref,
                      bc_ref, was_ref, wad_ref, ho_ref, t_ref):
    u0 = ua_ref[0, :, :] + ub_ref[0, :, :]
    u1 = ua_ref[1, :, :] + ub_ref[1, :, :]
    u2 = va_ref[0, :, :] + va_ref[1, :, :] + vb_ref[0, :, :] + vb_ref[1, :, :]
    hn = hn_ref[...]
    denom = u2[:, EIN:EIN + 1]
    rcp = jnp.where(denom > 0, 1.0 / jnp.maximum(denom, 1e-12), 0.0)
    wc = wc_ref[...]
    h_new = (jnp.dot(u0 * rcp, wc[0:128], preferred_element_type=jnp.float32)
             + jnp.dot(u1 * rcp, wc[128:256],
                       preferred_element_type=jnp.float32)
             + jnp.dot(u2[:, 0:EIN] * rcp, wc[256:272],
                       preferred_element_type=jnp.float32)
             + jnp.dot(hn, wc[272:272 + OUT],
                       preferred_element_type=jnp.float32)
             + bc_ref[...])
    h_out = jnp.where(denom > 0, h_new, hn)
    ho_ref[...] = h_out
    t_ref[:, 0:EIN] = jnp.dot(h_out, was_ref[...],
                              preferred_element_type=jnp.float32)
    t_ref[:, EIN:2 * EIN] = jnp.dot(h_out, wad_ref[...],
                                    preferred_element_type=jnp.float32)
    t_ref[:, 2 * EIN:128] = jnp.zeros_like(t_ref[:, 2 * EIN:128])


def _node_update(u01a, u01b, u2a, u2b, hn, Wc, b_c, Was, Wad):
    blk = 512
    grid = NPAD // blk
    full = lambda r, c: pl.BlockSpec((r, c), lambda i: (0, 0))
    stk = pl.BlockSpec((2, blk, 128), lambda i: (0, i, 0))
    return pl.pallas_call(
        _node_update_body,
        grid=(grid,),
        in_specs=[
            stk, stk, stk, stk,
            pl.BlockSpec((blk, OUT), lambda i: (i, 0)),
            full(2 * OUT + EIN, OUT),
            full(1, OUT),
            full(OUT, EIN),
            full(OUT, EIN),
        ],
        out_specs=[
            pl.BlockSpec((blk, OUT), lambda i: (i, 0)),
            pl.BlockSpec((blk, 128), lambda i: (i, 0)),
        ],
        out_shape=[
            jax.ShapeDtypeStruct((NPAD, OUT), jnp.float32),
            jax.ShapeDtypeStruct((NPAD, 128), jnp.float32),
        ],
    )(u01a, u01b, u2a, u2b, hn, Wc, b_c, Was, Wad)


_G2_PER_W = E // NW
_G2_NFULL = _G2_PER_W // BLK
_G2_TAIL = _G2_PER_W - _G2_NFULL * BLK
_G2_NG = 19


def _gather_pq_body(t_hbm, src_hbm, dst_hbm, wp_hbm, wout_hbm,
                    si_full, di_full,
                    rs0, rs1, rd0, rd1,
                    wp0, wp1,
                    rs_t, rd_t, wp_t, sem):
    wid = lax.axis_index("s") * NC + lax.axis_index("c")
    base = wid * _G2_PER_W
    rss = (rs0, rs1)
    rds = (rd0, rd1)
    wps = (wp0, wp1)

    pltpu.sync_copy(src_hbm.at[pl.ds(base, _G2_PER_W)], si_full)
    pltpu.sync_copy(dst_hbm.at[pl.ds(base, _G2_PER_W)], di_full)

    def compute_rows(n, rs_ref, rd_ref, wp_ref):
        def row(j, carry):
            wp_ref[j, :] = (rs_ref[j, pl.ds(0, EIN)]
                            + rd_ref[j, pl.ds(EIN, EIN)]
                            + wp_ref[j, :])
            return carry

        lax.fori_loop(0, n, row, 0)

    def do_set(goff, nk):
        hs = []
        for k in range(nk):
            loc = goff + k * BLK
            for cp in (
                pltpu.make_async_copy(
                    t_hbm.at[si_full.at[pl.ds(loc, BLK)]], rss[k], sem),
                pltpu.make_async_copy(
                    t_hbm.at[di_full.at[pl.ds(loc, BLK)]], rds[k], sem),
                pltpu.make_async_copy(
                    wp_hbm.at[pl.ds(base + loc, BLK)], wps[k], sem),
            ):
                cp.start()
                hs.append(cp)
        for cp in hs:
            cp.wait()
        for k in range(nk):
            compute_rows(BLK, rss[k], rds[k], wps[k])
        ws = []
        for k in range(nk):
            cp = pltpu.make_async_copy(
                wps[k], wout_hbm.at[pl.ds(base + goff + k * BLK, BLK)], sem)
            cp.start()
            ws.append(cp)
        for cp in ws:
            cp.wait()

    def group(g, carry):
        do_set(g * (2 * BLK), 2)
        return carry

    lax.fori_loop(0, _G2_NG, group, 0)
    do_set(_G2_NG * 2 * BLK, 1)
    toff = _G2_NFULL * BLK
    pltpu.sync_copy(t_hbm.at[si_full.at[pl.ds(toff, _G2_TAIL)]], rs_t)
    pltpu.sync_copy(t_hbm.at[di_full.at[pl.ds(toff, _G2_TAIL)]], rd_t)
    pltpu.sync_copy(wp_hbm.at[pl.ds(base + toff, _G2_TAIL)], wp_t)
    compute_rows(_G2_TAIL, rs_t, rd_t, wp_t)
    pltpu.sync_copy(wp_t, wout_hbm.at[pl.ds(base + toff, _G2_TAIL)])


def _gather_pq(T, src, dst, wpart):
    f = pl.kernel(
        _gather_pq_body,
        name="sc_g2",
        out_type=[jax.ShapeDtypeStruct((E, EIN), jnp.float32)],
        mesh=_mesh(),
        scratch_types=(
            [pltpu.VMEM((_G2_PER_W,), jnp.int32)] * 2
            + [pltpu.VMEM((BLK, 128), jnp.float32)] * 4
            + [pltpu.VMEM((BLK, EIN), jnp.float32)] * 2
            + [pltpu.VMEM((_G2_TAIL, 128), jnp.float32)] * 2
            + [pltpu.VMEM((_G2_TAIL, EIN), jnp.float32)] * 1
            + [pltpu.SemaphoreType.DMA]
        ),
    )
    (w_out,) = f(T, src, dst, wpart)
    return w_out


def kernel(h, edge_w, W_embed_node, W_attn_fc, W_inter_fuse, b_inter_fuse,
           W_embed_edge, W_edge_sf_atten, W_concentrate_h, b_concentrate_h,
           W_aggre, edge_index):
    src = edge_index[0].astype(jnp.int32)
    dst = edge_index[1].astype(jnp.int32)

    h_pad = jnp.pad(h, ((0, NPAD - N), (0, 0)))
    hn = _node_embed(h_pad, W_embed_node)

    src1, src2 = src[:EC1], src[EC1:]
    dst1, dst2 = dst[:EC1], dst[EC1:]
    hs1, hd1 = _gather_hn_1(hn, src1, dst1)
    hs2, hd2 = _gather_hn_2(hn, src2, dst2)

    b_if = b_inter_fuse.reshape(1, OUT)
    waw = W_aggre[2 * OUT:2 * OUT + EIN]
    wao = W_aggre[2 * OUT + EIN:]
    chain1 = _make_edge_chain(EC1, 0, False)
    chain2 = _make_edge_chain(EC2, EC1 // BLKE, True)
    sat1, p01_1, p2_1, wp1 = chain1(
        hs1, hd1, edge_w[:EC1], W_inter_fuse, b_if, W_embed_edge,
        W_edge_sf_atten, W_attn_fc, waw, wao)
    sat, p01_2, p2_2, wpart = chain2(
        hs2, hd2, edge_w[EC1:], W_inter_fuse, b_if, W_embed_edge,
        W_edge_sf_atten, W_attn_fc, waw, wao, sat1, wp1)

    zero_blk = jnp.zeros((ZROWS, 128), jnp.float32)
    u01a, u2a = _segment_scatter_1(p01_1, p2_1, dst1, zero_blk)
    u01b, u2b = _segment_scatter_2(p01_2, p2_2, dst2, zero_blk)

    h_out_pad, T = _node_update(
        u01a, u01b, u2a, u2b, hn, W_concentrate_h,
        b_concentrate_h.reshape(1, OUT), W_aggre[0:OUT], W_aggre[OUT:2 * OUT])

    w_out = _gather_pq(T, src, dst, wpart)

    return h_out_pad[:N], w_out, sat.T
